# Initial kernel scaffold; baseline (speedup 1.0000x reference)
import jax
import jax.numpy as jnp
from jax.experimental import pallas as pl


def kernel(cos_theta, cos_theta_aux, target):
    raise NotImplementedError("write your pallas kernel here")


# speedup vs baseline: 3.4072x; 3.4072x over previous
import functools

import jax
import jax.numpy as jnp
from jax.experimental import pallas as pl
from jax.experimental.pallas import tpu as pltpu

_S = 3.0
_M = 0.2


def _loss_kernel(x_ref, t_ref, out_ref, m_ref, s_ref, tv_ref, *, nblk, blk, C, B):
    k = pl.program_id(0)

    @pl.when(k == 0)
    def _init():
        m_ref[...] = jnp.full((B, 1), -jnp.inf, jnp.float32)
        s_ref[...] = jnp.zeros((B, 1), jnp.float32)
        tv_ref[...] = jnp.zeros((B, 1), jnp.float32)

    x = x_ref[...] * _S
    cols = jax.lax.broadcasted_iota(jnp.int32, (B, blk), 1) + k * blk
    x = jnp.where(cols < C, x, -jnp.inf)

    t = t_ref[...]
    tv_ref[...] += jnp.sum(jnp.where(cols == t, x, 0.0), axis=1, keepdims=True)

    bm = jnp.max(x, axis=1, keepdims=True)
    bs = jnp.sum(jnp.exp(x - bm), axis=1, keepdims=True)
    m_old = m_ref[...]
    m_new = jnp.maximum(m_old, bm)
    s_ref[...] = s_ref[...] * jnp.exp(m_old - m_new) + bs * jnp.exp(bm - m_new)
    m_ref[...] = m_new

    @pl.when(k == nblk - 1)
    def _finish():
        m = m_ref[...]
        s = s_ref[...]
        tv = tv_ref[...]
        out_t = tv - _S * _M
        s_c = s - jnp.exp(tv - m) + jnp.exp(out_t - m)
        lse = m + jnp.log(s_c)
        loss = lse - out_t
        out_ref[...] = (jnp.sum(loss) / B).reshape(1, 1)


def kernel(cos_theta, cos_theta_aux, target):
    B, C = cos_theta.shape
    blk = 2048
    nblk = pl.cdiv(C, blk)
    t2 = target.reshape(B, 1).astype(jnp.int32)
    out = pl.pallas_call(
        functools.partial(_loss_kernel, nblk=nblk, blk=blk, C=C, B=B),
        grid=(nblk,),
        in_specs=[
            pl.BlockSpec((B, blk), lambda k: (0, k)),
            pl.BlockSpec((B, 1), lambda k: (0, 0)),
        ],
        out_specs=pl.BlockSpec((1, 1), lambda k: (0, 0)),
        out_shape=jax.ShapeDtypeStruct((1, 1), jnp.float32),
        scratch_shapes=[
            pltpu.VMEM((B, 1), jnp.float32),
            pltpu.VMEM((B, 1), jnp.float32),
            pltpu.VMEM((B, 1), jnp.float32),
        ],
    )(cos_theta, t2)
    return out[0, 0]
